# Initial kernel scaffold; baseline (speedup 1.0000x reference)
import jax
import jax.numpy as jnp
from jax.experimental import pallas as pl


def kernel(x, edge_index, W_lin1, Wc1, Wc2, Wc3, W_lin2):
    raise NotImplementedError("write your pallas kernel here")


# speedup vs baseline: 2.6698x; 2.6698x over previous
import functools

import numpy as np
import jax
import jax.numpy as jnp
from jax import lax
from jax.experimental import pallas as pl
from jax.experimental.pallas import tpu as pltpu
from jax.experimental.pallas import tpu_sc as plsc

_N = 10000
_NP = 10240
_E = 320000
_F = 128
_NC = 2
_NS = 16
_NW = _NC * _NS
_CHUNK = 80
_NCH = _E // (_NW * _CHUNK)
_RPT = _NP // _NS
_BLK = 128
_DLANE = 16

_ALPHA = 0.1
_BETAS = tuple(float(np.log(1.0 / k + 1.0)) for k in (1, 2, 3))

_MESH = dict(core_axis_name="c", subcore_axis_name="s")


def _make_edge_pass(feat):
    @functools.partial(
        pl.kernel,
        mesh=plsc.VectorSubcoreMesh(**_MESH),
        out_type=jax.ShapeDtypeStruct((_NC, _NP, feat), jnp.float32),
        scratch_types=[
            pltpu.VMEM((_CHUNK,), jnp.int32),
            pltpu.VMEM((_CHUNK,), jnp.int32),
            pltpu.VMEM((_CHUNK, feat), jnp.float32),
            pltpu.VMEM((_BLK, feat), jnp.float32),
            pltpu.VMEM_SHARED((_NP, feat), jnp.float32),
            pltpu.SemaphoreType.DMA,
        ],
    )
    def _pass(table_hbm, src_hbm, dst_hbm, out_hbm,
              idx_s, idx_d, rows_v, zv, agg_sh, sem):
        c = lax.axis_index("c")
        s = lax.axis_index("s")
        w = s * _NC + c
        r0 = s * _RPT

        def _fill_zero(i, carry):
            for k in range(feat // 16):
                zv[i, pl.ds(16 * k, 16)] = jnp.zeros((16,), jnp.float32)
            return carry

        lax.fori_loop(0, _BLK, _fill_zero, 0)

        def _zero_blk(st, carry):
            pltpu.sync_copy(zv, agg_sh.at[pl.ds(r0 + st * _BLK, _BLK)])
            return carry

        lax.fori_loop(0, _RPT // _BLK, _zero_blk, 0)
        plsc.subcore_barrier()

        def _chunk(ch, carry):
            pltpu.sync_copy(src_hbm.at[w, ch], idx_s)
            pltpu.sync_copy(dst_hbm.at[w, ch], idx_d)
            pltpu.async_copy(table_hbm.at[idx_s], rows_v, sem).wait()
            pltpu.sync_copy(rows_v, agg_sh.at[idx_d], add=True)
            return carry

        lax.fori_loop(0, _NCH, _chunk, 0)
        plsc.subcore_barrier()

        def _out_blk(st, carry):
            b0 = r0 + st * _BLK
            pltpu.sync_copy(agg_sh.at[pl.ds(b0, _BLK)], zv)
            pltpu.sync_copy(zv, out_hbm.at[c, pl.ds(b0, _BLK)])
            return carry

        lax.fori_loop(0, _RPT // _BLK, _out_blk, 0)

    return _pass


_sc_edge_pass = _make_edge_pass(_F)


_ROWS = 1024


def _tc_prep(x, w_lin1, cnt):
    def body(x_ref, w_ref, cnt_ref, x0_ref, g1_ref, ns_ref, nd_ref):
        cs = cnt_ref[0, 0, :, 0:1] + cnt_ref[1, 0, :, 0:1]
        cd = cnt_ref[0, 1, :, 0:1] + cnt_ref[1, 1, :, 0:1]
        ns = lax.rsqrt(jnp.maximum(cs, 1.0))
        nd = lax.rsqrt(jnp.maximum(cd, 1.0))
        x0 = jnp.dot(x_ref[...], w_ref[...], preferred_element_type=jnp.float32,
                     precision=lax.Precision.HIGHEST)
        x0_ref[...] = x0
        g1_ref[...] = x0 * ns
        ns_ref[...] = ns
        nd_ref[...] = nd

    return pl.pallas_call(
        body,
        grid=(_NP // _ROWS,),
        in_specs=[
            pl.BlockSpec((_ROWS, _F), lambda i: (i, 0)),
            pl.BlockSpec((_F, _F), lambda i: (0, 0)),
            pl.BlockSpec((_NC, 2, _ROWS, _F), lambda i: (0, 0, i, 0)),
        ],
        out_specs=[
            pl.BlockSpec((_ROWS, _F), lambda i: (i, 0)),
            pl.BlockSpec((_ROWS, _F), lambda i: (i, 0)),
            pl.BlockSpec((_ROWS, 1), lambda i: (i, 0)),
            pl.BlockSpec((_ROWS, 1), lambda i: (i, 0)),
        ],
        out_shape=[
            jax.ShapeDtypeStruct((_NP, _F), jnp.float32),
            jax.ShapeDtypeStruct((_NP, _F), jnp.float32),
            jax.ShapeDtypeStruct((_NP, 1), jnp.float32),
            jax.ShapeDtypeStruct((_NP, 1), jnp.float32),
        ],
    )(x, w_lin1, cnt)


def _tc_layer(p, x0, ns, nd, w, beta):
    def body(p_ref, x0_ref, ns_ref, nd_ref, w_ref, o_ref):
        f = ((1.0 - _ALPHA) * (p_ref[0] + p_ref[1]) * nd_ref[...]
             + _ALPHA * x0_ref[...])
        rst = (1.0 - beta) * f + beta * jnp.dot(
            f, w_ref[...], preferred_element_type=jnp.float32,
            precision=lax.Precision.HIGHEST)
        o_ref[...] = rst * ns_ref[...]

    return pl.pallas_call(
        body,
        grid=(_NP // _ROWS,),
        in_specs=[
            pl.BlockSpec((_NC, _ROWS, _F), lambda i: (0, i, 0)),
            pl.BlockSpec((_ROWS, _F), lambda i: (i, 0)),
            pl.BlockSpec((_ROWS, 1), lambda i: (i, 0)),
            pl.BlockSpec((_ROWS, 1), lambda i: (i, 0)),
            pl.BlockSpec((_F, _F), lambda i: (0, 0)),
        ],
        out_specs=pl.BlockSpec((_ROWS, _F), lambda i: (i, 0)),
        out_shape=jax.ShapeDtypeStruct((_NP, _F), jnp.float32),
    )(p, x0, ns, nd, w)


def _tc_final(p, x0, nd, w3, w_lin2, beta):
    def body(p_ref, x0_ref, nd_ref, w3_ref, w2_ref, o_ref):
        f = ((1.0 - _ALPHA) * (p_ref[0] + p_ref[1]) * nd_ref[...]
             + _ALPHA * x0_ref[...])
        rst = (1.0 - beta) * f + beta * jnp.dot(
            f, w3_ref[...], preferred_element_type=jnp.float32,
            precision=lax.Precision.HIGHEST)
        o_ref[...] = jnp.dot(rst, w2_ref[...], preferred_element_type=jnp.float32,
                             precision=lax.Precision.HIGHEST)

    return pl.pallas_call(
        body,
        grid=(_NP // _ROWS,),
        in_specs=[
            pl.BlockSpec((_NC, _ROWS, _F), lambda i: (0, i, 0)),
            pl.BlockSpec((_ROWS, _F), lambda i: (i, 0)),
            pl.BlockSpec((_ROWS, 1), lambda i: (i, 0)),
            pl.BlockSpec((_F, _F), lambda i: (0, 0)),
            pl.BlockSpec((_F, _F), lambda i: (0, 0)),
        ],
        out_specs=pl.BlockSpec((_ROWS, _F), lambda i: (i, 0)),
        out_shape=jax.ShapeDtypeStruct((_NP, _F), jnp.float32),
    )(p, x0, nd, w3, w_lin2)


def kernel(x, edge_index, W_lin1, Wc1, Wc2, Wc3, W_lin2):
    src3 = edge_index[0].reshape(_NW, _NCH, _CHUNK)
    dst3 = edge_index[1].reshape(_NW, _NCH, _CHUNK)
    xp = jnp.pad(x, ((0, _NP - _N), (0, 0)))
    ones_t = jnp.ones((_NP, _F), jnp.float32)
    cnt_src = _sc_edge_pass(ones_t, dst3, src3)
    cnt_dst = _sc_edge_pass(ones_t, src3, dst3)
    cnt = jnp.stack([cnt_src, cnt_dst], axis=1)
    x0, g1, ns, nd = _tc_prep(xp, W_lin1, cnt)
    p = _sc_edge_pass(g1, src3, dst3)
    g2 = _tc_layer(p, x0, ns, nd, Wc1, _BETAS[0])
    p = _sc_edge_pass(g2, src3, dst3)
    g3 = _tc_layer(p, x0, ns, nd, Wc2, _BETAS[1])
    p = _sc_edge_pass(g3, src3, dst3)
    return _tc_final(p, x0, nd, Wc3, W_lin2, _BETAS[2])[:_N]
